# baseline (speedup 1.0000x reference)
import functools

import jax
import jax.numpy as jnp
from jax import lax
from jax.experimental import pallas as pl
from jax.experimental.pallas import tpu as pltpu
from jax.experimental.pallas import tpu_sc as plsc

N = 10000
D = 256
H = 128
E = 160000
P = 81
NC = 2
NS = 16
EB = 96
NB = 112
NQ = EB // 16
EPAD = NS * NB * EB
RT = 624
RCHUNK = 16


def _tc_body(rf_ref, wc_ref, wg_ref, tself_ref, tsub_ref, tobj_ref, lgp_ref):
    rf = rf_ref[...]
    t = jnp.dot(rf, wc_ref[...], preferred_element_type=jnp.float32)
    lgp_ref[...] = jnp.dot(rf, wg_ref[...], preferred_element_type=jnp.float32)
    tself_ref[0] = t[:, 0:H]
    tself_ref[1] = t[:, H:2 * H]
    tsub_ref[0] = t[:, 2 * H:3 * H]
    tsub_ref[1] = t[:, 3 * H:4 * H]
    tobj_ref[0] = t[:, 4 * H:5 * H]
    tobj_ref[1] = t[:, 5 * H:6 * H]


def _tc_transform(region_feats, W_conv, wg_pad):
    R = 400
    return pl.pallas_call(
        _tc_body,
        grid=(N // R,),
        in_specs=[
            pl.BlockSpec((R, D), lambda i: (i, 0)),
            pl.BlockSpec((D, 3 * D), lambda i: (0, 0)),
            pl.BlockSpec((D, H), lambda i: (0, 0)),
        ],
        out_specs=[
            pl.BlockSpec((NC, R, H), lambda i: (0, i, 0)),
            pl.BlockSpec((NC, R, H), lambda i: (0, i, 0)),
            pl.BlockSpec((NC, R, H), lambda i: (0, i, 0)),
            pl.BlockSpec((R, H), lambda i: (i, 0)),
        ],
        out_shape=[
            jax.ShapeDtypeStruct((NC, N, H), jnp.float32),
            jax.ShapeDtypeStruct((NC, N, H), jnp.float32),
            jax.ShapeDtypeStruct((NC, N, H), jnp.float32),
            jax.ShapeDtypeStruct((N, H), jnp.float32),
        ],
    )(region_feats, W_conv, wg_pad)


def _sigmoid16(x):
    return 1.0 / (1.0 + jnp.exp(-x))


GB = 8


def _sc_body(tself_ref, tsub_ref, tobj_ref, g0_ref, g12_ref,
             sj_ref, ob_ref, pr_ref, blab_ref, gb_ref,
             out_ref,
             agg, sj2, ob2, pr2, g12v, blabv, gbv, sgv,
             gatevA, gatevB, adjvA, adjvB, dstB, prB,
             msgA, msgB, semA, semB):
    c = lax.axis_index("c")
    s = lax.axis_index("s")
    cN = c * N

    pltpu.sync_copy(g12_ref, g12v)
    pltpu.sync_copy(gb_ref, gbv)
    pltpu.sync_copy(blab_ref.at[pl.ds(c * 88, 88)], blabv)

    base = s * RT
    nchunks = jnp.where(s == NS - 1, 40, 39)

    pltpu.sync_copy(g0_ref.at[pl.ds(base, 640)], sgv)

    def _sig_body(k, carry):
        x = sgv[pl.ds(k * 16, 16)]
        sgv[pl.ds(k * 16, 16)] = _sigmoid16(x)
        return carry

    lax.fori_loop(0, 40, _sig_body, 0)

    rowbuf = msgA.at[pl.ds(0, RCHUNK)]

    def _ph1_body(k, carry):
        r0 = base + k * RCHUNK
        pltpu.sync_copy(tself_ref.at[pl.ds(cN + r0, RCHUNK)], rowbuf)
        sgvec = sgv[pl.ds(k * RCHUNK, RCHUNK)]
        for i in range(RCHUNK):
            sg = sgvec[i]
            for t in range(8):
                sl = pl.ds(t * 16, 16)
                msgA[i, sl] = msgA[i, sl] * sg
        pltpu.sync_copy(rowbuf, agg.at[pl.ds(r0, RCHUNK)])
        return carry

    lax.fori_loop(0, nchunks, _ph1_body, 0)
    plsc.subcore_barrier()

    def _stage_group(g):
        row0 = s * NB + g * GB
        pltpu.sync_copy(sj_ref.at[pl.ds(row0, GB)], sj2)
        pltpu.sync_copy(ob_ref.at[pl.ds(row0, GB)], ob2)
        pltpu.sync_copy(pr_ref.at[pl.ds(row0, GB)], pr2)

    def _unpack_gate(packed, low_half):
        if low_half:
            bits = jnp.left_shift(packed, 16)
        else:
            bits = jnp.bitwise_and(packed, jnp.int32(-65536))
        return plsc.bitcast(bits, jnp.float32)

    def _prep(rb, src2, low_half, adjv, gatev, copy_b):
        def _gate_body(t, carry):
            sl = pl.ds(t * 16, 16)
            src16 = jnp.minimum(src2[rb, sl], N - 1)
            pr16 = pr2[rb, sl]
            adjv[sl] = src16 + cN
            if copy_b:
                dstB[sl] = ob2[rb, sl]
                prB[sl] = pr16
            packed = plsc.load_gather(g12v, [src16])
            x = _unpack_gate(packed, low_half) + plsc.load_gather(gbv, [pr16])
            gatev[sl] = _sigmoid16(x)
            return carry

        lax.fori_loop(0, NQ, _gate_body, 0)

    def _scale(msg, gatev, pget):
        def _scale_body(q, carry):
            e0 = q * 16
            gvec = gatev[pl.ds(e0, 16)]
            pvec = pget(pl.ds(e0, 16))

            def _loads(i):
                p = pvec[i]
                m = [msg[e0 + i, pl.ds(t * 16, 16)] for t in range(8)]
                bw = [blabv[p, pl.ds(u * 16, 16)] for u in range(4)]
                return m, bw

            cur = _loads(0)
            for i in range(16):
                nxt = _loads(i + 1) if i + 1 < 16 else None
                g = gvec[i]
                m, bw = cur
                for t in range(8):
                    bl = _unpack_gate(bw[t % 4], t < 4)
                    msg[e0 + i, pl.ds(t * 16, 16)] = (m[t] + bl) * g
                cur = nxt
            return carry

        lax.fori_loop(0, NQ, _scale_body, 0)

    def _batch_body(b, carry):
        rb = lax.rem(b, GB)
        rb1 = lax.rem(b + 1, GB)
        _prep(rb, sj2, False, adjvB, gatevB, True)
        pltpu.async_copy(tobj_ref.at[adjvB], msgB, semB)
        pltpu.make_async_copy(tsub_ref.at[adjvA], msgA, semA).wait()
        _scale(msgA, gatevA, lambda sl: pr2[rb, sl])
        pltpu.sync_copy(msgA, agg.at[sj2.at[rb]], add=True)
        @pl.when(jnp.logical_and(b + 1 < NB, rb1 == 0))
        def _():
            _stage_group((b + 1) // GB)
        @pl.when(b + 1 < NB)
        def _():
            _prep(rb1, ob2, True, adjvA, gatevA, False)
            pltpu.async_copy(tsub_ref.at[adjvA], msgA, semA)
        pltpu.make_async_copy(tobj_ref.at[adjvB], msgB, semB).wait()
        _scale(msgB, gatevB, lambda sl: prB[sl])
        pltpu.sync_copy(msgB, agg.at[dstB], add=True)
        return carry

    _stage_group(0)
    _prep(0, ob2, True, adjvA, gatevA, False)
    pltpu.async_copy(tsub_ref.at[adjvA], msgA, semA)
    lax.fori_loop(0, NB, _batch_body, 0)
    plsc.subcore_barrier()

    def _ph3_body(k, carry):
        r0 = base + k * RCHUNK
        pltpu.sync_copy(agg.at[pl.ds(r0, RCHUNK)], rowbuf)
        for i in range(RCHUNK):
            for t in range(8):
                sl = pl.ds(t * 16, 16)
                msgA[i, sl] = jnp.maximum(msgA[i, sl], 0.0)
        pltpu.sync_copy(rowbuf, out_ref.at[pl.ds(cN + r0, RCHUNK)])
        return carry

    lax.fori_loop(0, nchunks, _ph3_body, 0)


_SC_SCRATCH = [
    pltpu.VMEM_SHARED((N + 8, H), jnp.float32),
    pltpu.VMEM((GB, EB), jnp.int32),
    pltpu.VMEM((GB, EB), jnp.int32),
    pltpu.VMEM((GB, EB), jnp.int32),
    pltpu.VMEM((N,), jnp.int32),
    pltpu.VMEM((88, H // 2), jnp.int32),
    pltpu.VMEM((88,), jnp.float32),
    pltpu.VMEM((640,), jnp.float32),
    pltpu.VMEM((EB,), jnp.float32),
    pltpu.VMEM((EB,), jnp.float32),
    pltpu.VMEM((EB,), jnp.int32),
    pltpu.VMEM((EB,), jnp.int32),
    pltpu.VMEM((EB,), jnp.int32),
    pltpu.VMEM((EB,), jnp.int32),
    pltpu.VMEM((EB, H), jnp.float32),
    pltpu.VMEM((EB, H), jnp.float32),
    pltpu.SemaphoreType.DMA,
    pltpu.SemaphoreType.DMA,
]


def _sc_aggregate(tself_flat, tsub_flat, tobj_flat, g0, g12,
                  sj2d, ob2d, pr2d, blab2, gbp):
    mesh = plsc.VectorSubcoreMesh(core_axis_name="c", subcore_axis_name="s")
    fn = pl.kernel(
        _sc_body,
        out_type=jax.ShapeDtypeStruct((NC * N, H), jnp.float32),
        mesh=mesh,
        scratch_types=_SC_SCRATCH,
        compiler_params=pltpu.CompilerParams(needs_layout_passes=False),
    )
    return fn(tself_flat, tsub_flat, tobj_flat, g0, g12,
              sj2d, ob2d, pr2d, blab2, gbp)


def kernel(region_feats, rels, pred_classes, W_conv, W_g, b_lab, b_glab):
    wg_pad = jnp.zeros((D, H), jnp.float32).at[:, :3].set(W_g)
    tself_h, tsub_h, tobj_h, lgp = _tc_transform(region_feats, W_conv, wg_pad)

    tself_flat = tself_h.reshape(NC * N, H)
    tsub_flat = tsub_h.reshape(NC * N, H)
    tobj_flat = tobj_h.reshape(NC * N, H)
    g0 = lgp[:, 0]
    g1b = lax.bitcast_convert_type(lgp[:, 1].astype(jnp.bfloat16),
                                   jnp.uint16).astype(jnp.uint32)
    g2b = lax.bitcast_convert_type(lgp[:, 2].astype(jnp.bfloat16),
                                   jnp.uint16).astype(jnp.uint32)
    g12 = lax.bitcast_convert_type(g1b | (g2b << 16), jnp.int32)

    subj = rels[:, 1]
    obj = rels[:, 2]
    pad = EPAD - E
    padv = jnp.full((pad,), N, jnp.int32)
    sj2d = jnp.concatenate([subj, padv]).reshape(NS * NB, EB)
    ob2d = jnp.concatenate([obj, padv]).reshape(NS * NB, EB)
    pr2d = jnp.concatenate([pred_classes, jnp.zeros((pad,), jnp.int32)]
                           ).reshape(NS * NB, EB)
    def _pack_half(hm):
        lo = lax.bitcast_convert_type(hm[:, :64].astype(jnp.bfloat16),
                                      jnp.uint16).astype(jnp.uint32)
        hi = lax.bitcast_convert_type(hm[:, 64:].astype(jnp.bfloat16),
                                      jnp.uint16).astype(jnp.uint32)
        packed = lax.bitcast_convert_type(lo | (hi << 16), jnp.int32)
        return jnp.pad(packed, ((0, 7), (0, 0)))
    blab2 = jnp.concatenate([_pack_half(b_lab[:, :H]),
                             _pack_half(b_lab[:, H:])], axis=0)
    gbp = jnp.pad(b_glab[:, 0], (0, 7))

    out_flat = _sc_aggregate(tself_flat, tsub_flat, tobj_flat, g0, g12,
                             sj2d, ob2d, pr2d, blab2, gbp)
    out2 = out_flat.reshape(NC, N, H)
    return jnp.concatenate([out2[0], out2[1]], axis=1)


# speedup vs baseline: 1.4883x; 1.4883x over previous
import functools

import jax
import jax.numpy as jnp
from jax import lax
from jax.experimental import pallas as pl
from jax.experimental.pallas import tpu as pltpu
from jax.experimental.pallas import tpu_sc as plsc

N = 10000
D = 256
H = 128
E = 160000
P = 81
NC = 2
NS = 16
EB = 128
NB = 80
NQ = EB // 16
EPAD = NS * NB * EB
RT = 624
RCHUNK = 16


def _tc_body(rf_ref, wc_ref, wg_ref, tself_ref, tsub_ref, tobj_ref, lgp_ref):
    rf = rf_ref[...]
    t = jnp.dot(rf, wc_ref[...], preferred_element_type=jnp.float32)
    lgp_ref[...] = jnp.dot(rf, wg_ref[...], preferred_element_type=jnp.float32)
    tself_ref[0] = t[:, 0:H]
    tself_ref[1] = t[:, H:2 * H]
    tsub_ref[0] = t[:, 2 * H:3 * H]
    tsub_ref[1] = t[:, 3 * H:4 * H]
    tobj_ref[0] = t[:, 4 * H:5 * H]
    tobj_ref[1] = t[:, 5 * H:6 * H]


def _tc_transform(region_feats, W_conv, wg_pad):
    R = 400
    return pl.pallas_call(
        _tc_body,
        grid=(N // R,),
        in_specs=[
            pl.BlockSpec((R, D), lambda i: (i, 0)),
            pl.BlockSpec((D, 3 * D), lambda i: (0, 0)),
            pl.BlockSpec((D, H), lambda i: (0, 0)),
        ],
        out_specs=[
            pl.BlockSpec((NC, R, H), lambda i: (0, i, 0)),
            pl.BlockSpec((NC, R, H), lambda i: (0, i, 0)),
            pl.BlockSpec((NC, R, H), lambda i: (0, i, 0)),
            pl.BlockSpec((R, H), lambda i: (i, 0)),
        ],
        out_shape=[
            jax.ShapeDtypeStruct((NC, N, H), jnp.float32),
            jax.ShapeDtypeStruct((NC, N, H), jnp.float32),
            jax.ShapeDtypeStruct((NC, N, H), jnp.float32),
            jax.ShapeDtypeStruct((N, H), jnp.float32),
        ],
    )(region_feats, W_conv, wg_pad)


def _sigmoid16(x):
    return 1.0 / (1.0 + jnp.exp(-x))


GB = 8


def _sc_body(tself_ref, tsub_ref, tobj_ref, g0_ref, g12_ref,
             sj_ref, ob_ref, pr_ref, blab_ref, gb_ref,
             out_ref,
             agg, sj2, ob2, pr2, g12v, blabv, gbv, sgv,
             gatev, adjv, msg, sem):
    c = lax.axis_index("c")
    s = lax.axis_index("s")
    cN = c * N

    pltpu.sync_copy(g12_ref, g12v)
    pltpu.sync_copy(gb_ref, gbv)
    pltpu.sync_copy(blab_ref.at[pl.ds(c * 88, 88)], blabv)

    base = s * RT
    nchunks = jnp.where(s == NS - 1, 40, 39)

    pltpu.sync_copy(g0_ref.at[pl.ds(base, 640)], sgv)

    def _sig_body(k, carry):
        x = sgv[pl.ds(k * 16, 16)]
        sgv[pl.ds(k * 16, 16)] = _sigmoid16(x)
        return carry

    lax.fori_loop(0, 40, _sig_body, 0)

    rowbuf = msg.at[pl.ds(0, RCHUNK)]

    def _ph1_body(k, carry):
        r0 = base + k * RCHUNK
        pltpu.sync_copy(tself_ref.at[pl.ds(cN + r0, RCHUNK)], rowbuf)
        sgvec = sgv[pl.ds(k * RCHUNK, RCHUNK)]
        for i in range(RCHUNK):
            sg = sgvec[i]
            for t in range(8):
                sl = pl.ds(t * 16, 16)
                msg[i, sl] = msg[i, sl] * sg
        pltpu.sync_copy(rowbuf, agg.at[pl.ds(r0, RCHUNK)])
        return carry

    lax.fori_loop(0, nchunks, _ph1_body, 0)
    plsc.subcore_barrier()

    def _stage_group(g):
        row0 = s * NB + g * GB
        pltpu.sync_copy(sj_ref.at[pl.ds(row0, GB)], sj2)
        pltpu.sync_copy(ob_ref.at[pl.ds(row0, GB)], ob2)
        pltpu.sync_copy(pr_ref.at[pl.ds(row0, GB)], pr2)

    def _unpack_gate(packed, low_half):
        if low_half:
            bits = jnp.left_shift(packed, 16)
        else:
            bits = jnp.bitwise_and(packed, jnp.int32(-65536))
        return plsc.bitcast(bits, jnp.float32)

    def _direction(rb, src2, low_half, table_ref, dst2):
        def _gate_body(t, carry):
            sl = pl.ds(t * 16, 16)
            src16 = jnp.minimum(src2[rb, sl], N - 1)
            pr16 = pr2[rb, sl]
            adjv[sl] = src16 + cN
            packed = plsc.load_gather(g12v, [src16])
            x = _unpack_gate(packed, low_half) + plsc.load_gather(gbv, [pr16])
            gatev[sl] = _sigmoid16(x)
            return carry

        lax.fori_loop(0, NQ, _gate_body, 0)
        pltpu.async_copy(table_ref.at[adjv], msg, sem).wait()

        def _scale_body(q, carry):
            e0 = q * 16
            gvec = gatev[pl.ds(e0, 16)]
            pvec = pr2[rb, pl.ds(e0, 16)]

            def _loads(i):
                p = pvec[i]
                m = [msg[e0 + i, pl.ds(t * 16, 16)] for t in range(8)]
                bw = [blabv[p, pl.ds(u * 16, 16)] for u in range(4)]
                return m, bw

            cur = _loads(0)
            for i in range(16):
                nxt = _loads(i + 1) if i + 1 < 16 else None
                g = gvec[i]
                m, bw = cur
                for t in range(8):
                    bl = _unpack_gate(bw[t % 4], t < 4)
                    msg[e0 + i, pl.ds(t * 16, 16)] = (m[t] + bl) * g
                cur = nxt
            return carry

        lax.fori_loop(0, NQ, _scale_body, 0)
        pltpu.sync_copy(msg, agg.at[dst2.at[rb]], add=True)

    def _group_body(grp, carry):
        _stage_group(grp)

        def _pair_body(rb, carry2):
            _direction(rb, ob2, True, tsub_ref, sj2)
            _direction(rb, sj2, False, tobj_ref, ob2)
            return carry2

        lax.fori_loop(0, GB, _pair_body, 0)
        return carry

    lax.fori_loop(0, NB // GB, _group_body, 0)
    plsc.subcore_barrier()

    def _ph3_body(k, carry):
        r0 = base + k * RCHUNK
        pltpu.sync_copy(agg.at[pl.ds(r0, RCHUNK)], rowbuf)
        for i in range(RCHUNK):
            for t in range(8):
                sl = pl.ds(t * 16, 16)
                msg[i, sl] = jnp.maximum(msg[i, sl], 0.0)
        pltpu.sync_copy(rowbuf, out_ref.at[pl.ds(cN + r0, RCHUNK)])
        return carry

    lax.fori_loop(0, nchunks, _ph3_body, 0)


_SC_SCRATCH = [
    pltpu.VMEM_SHARED((N + 8, H), jnp.float32),
    pltpu.VMEM((GB, EB), jnp.int32),
    pltpu.VMEM((GB, EB), jnp.int32),
    pltpu.VMEM((GB, EB), jnp.int32),
    pltpu.VMEM((N,), jnp.int32),
    pltpu.VMEM((88, H // 2), jnp.int32),
    pltpu.VMEM((88,), jnp.float32),
    pltpu.VMEM((640,), jnp.float32),
    pltpu.VMEM((EB,), jnp.float32),
    pltpu.VMEM((EB,), jnp.int32),
    pltpu.VMEM((EB, H), jnp.float32),
    pltpu.SemaphoreType.DMA,
]


def _sc_aggregate(tself_flat, tsub_flat, tobj_flat, g0, g12,
                  sj2d, ob2d, pr2d, blab2, gbp):
    mesh = plsc.VectorSubcoreMesh(core_axis_name="c", subcore_axis_name="s")
    fn = pl.kernel(
        _sc_body,
        out_type=jax.ShapeDtypeStruct((NC * N, H), jnp.float32),
        mesh=mesh,
        scratch_types=_SC_SCRATCH,
        compiler_params=pltpu.CompilerParams(needs_layout_passes=False),
    )
    return fn(tself_flat, tsub_flat, tobj_flat, g0, g12,
              sj2d, ob2d, pr2d, blab2, gbp)


def kernel(region_feats, rels, pred_classes, W_conv, W_g, b_lab, b_glab):
    wg_pad = jnp.zeros((D, H), jnp.float32).at[:, :3].set(W_g)
    tself_h, tsub_h, tobj_h, lgp = _tc_transform(region_feats, W_conv, wg_pad)

    tself_flat = tself_h.reshape(NC * N, H)
    tsub_flat = tsub_h.reshape(NC * N, H)
    tobj_flat = tobj_h.reshape(NC * N, H)
    g0 = lgp[:, 0]
    g1b = lax.bitcast_convert_type(lgp[:, 1].astype(jnp.bfloat16),
                                   jnp.uint16).astype(jnp.uint32)
    g2b = lax.bitcast_convert_type(lgp[:, 2].astype(jnp.bfloat16),
                                   jnp.uint16).astype(jnp.uint32)
    g12 = lax.bitcast_convert_type(g1b | (g2b << 16), jnp.int32)

    subj = rels[:, 1]
    obj = rels[:, 2]
    pad = EPAD - E
    padv = jnp.full((pad,), N, jnp.int32)
    sj2d = jnp.concatenate([subj, padv]).reshape(NS * NB, EB)
    ob2d = jnp.concatenate([obj, padv]).reshape(NS * NB, EB)
    pr2d = jnp.concatenate([pred_classes, jnp.zeros((pad,), jnp.int32)]
                           ).reshape(NS * NB, EB)
    def _pack_half(hm):
        lo = lax.bitcast_convert_type(hm[:, :64].astype(jnp.bfloat16),
                                      jnp.uint16).astype(jnp.uint32)
        hi = lax.bitcast_convert_type(hm[:, 64:].astype(jnp.bfloat16),
                                      jnp.uint16).astype(jnp.uint32)
        packed = lax.bitcast_convert_type(lo | (hi << 16), jnp.int32)
        return jnp.pad(packed, ((0, 7), (0, 0)))
    blab2 = jnp.concatenate([_pack_half(b_lab[:, :H]),
                             _pack_half(b_lab[:, H:])], axis=0)
    gbp = jnp.pad(b_glab[:, 0], (0, 7))

    out_flat = _sc_aggregate(tself_flat, tsub_flat, tobj_flat, g0, g12,
                             sj2d, ob2d, pr2d, blab2, gbp)
    out2 = out_flat.reshape(NC, N, H)
    return jnp.concatenate([out2[0], out2[1]], axis=1)


# speedup vs baseline: 1.9310x; 1.2975x over previous
import functools

import jax
import jax.numpy as jnp
from jax import lax
from jax.experimental import pallas as pl
from jax.experimental.pallas import tpu as pltpu
from jax.experimental.pallas import tpu_sc as plsc

N = 10000
D = 256
H = 128
E = 160000
P = 81
NC = 2
NS = 16
EB = 128
NB = 80
NQ = EB // 16
EPAD = NS * NB * EB
RT = 624
RCHUNK = 16


def _pack_cols(x):
    lo = lax.bitcast_convert_type(x[:, :64].astype(jnp.bfloat16),
                                  jnp.uint16).astype(jnp.uint32)
    hi = lax.bitcast_convert_type(x[:, 64:].astype(jnp.bfloat16),
                                  jnp.uint16).astype(jnp.uint32)
    return lax.bitcast_convert_type(lo | (hi << 16), jnp.int32)


def _tc_body(rf_ref, wc_ref, wg_ref, tself_ref, tsub_ref, tobj_ref, lgp_ref):
    rf = rf_ref[...]
    t = jnp.dot(rf, wc_ref[...], preferred_element_type=jnp.float32)
    lgp_ref[...] = jnp.dot(rf, wg_ref[...], preferred_element_type=jnp.float32)
    tself_ref[0] = t[:, 0:H]
    tself_ref[1] = t[:, H:2 * H]
    tsub_ref[0] = _pack_cols(t[:, 2 * H:3 * H])
    tsub_ref[1] = _pack_cols(t[:, 3 * H:4 * H])
    tobj_ref[0] = _pack_cols(t[:, 4 * H:5 * H])
    tobj_ref[1] = _pack_cols(t[:, 5 * H:6 * H])


def _tc_transform(region_feats, W_conv, wg_pad):
    R = 400
    return pl.pallas_call(
        _tc_body,
        grid=(N // R,),
        in_specs=[
            pl.BlockSpec((R, D), lambda i: (i, 0)),
            pl.BlockSpec((D, 3 * D), lambda i: (0, 0)),
            pl.BlockSpec((D, H), lambda i: (0, 0)),
        ],
        out_specs=[
            pl.BlockSpec((NC, R, H), lambda i: (0, i, 0)),
            pl.BlockSpec((NC, R, H // 2), lambda i: (0, i, 0)),
            pl.BlockSpec((NC, R, H // 2), lambda i: (0, i, 0)),
            pl.BlockSpec((R, H), lambda i: (i, 0)),
        ],
        out_shape=[
            jax.ShapeDtypeStruct((NC, N, H), jnp.float32),
            jax.ShapeDtypeStruct((NC, N, H // 2), jnp.int32),
            jax.ShapeDtypeStruct((NC, N, H // 2), jnp.int32),
            jax.ShapeDtypeStruct((N, H), jnp.float32),
        ],
    )(region_feats, W_conv, wg_pad)


def _sigmoid16(x):
    return 1.0 / (1.0 + jnp.exp(-x))


GB = 8


def _sc_body(tself_ref, tsub_ref, tobj_ref, g0_ref, g12_ref,
             sj_ref, ob_ref, pr_ref, blab_ref, gb_ref,
             out_ref,
             agg, sj2, ob2, pr2, g12v, blabv, gbv, sgv,
             gatev, adjv, msgp, scbuf, sem):
    c = lax.axis_index("c")
    s = lax.axis_index("s")
    cN = c * N

    pltpu.sync_copy(g12_ref, g12v)
    pltpu.sync_copy(gb_ref, gbv)
    pltpu.sync_copy(blab_ref.at[pl.ds(c * 88, 88)], blabv)

    base = s * RT
    nchunks = jnp.where(s == NS - 1, 40, 39)

    pltpu.sync_copy(g0_ref.at[pl.ds(base, 640)], sgv)

    def _sig_body(k, carry):
        x = sgv[pl.ds(k * 16, 16)]
        sgv[pl.ds(k * 16, 16)] = _sigmoid16(x)
        return carry

    lax.fori_loop(0, 40, _sig_body, 0)

    rowbuf = scbuf.at[pl.ds(0, RCHUNK)]

    def _ph1_body(k, carry):
        r0 = base + k * RCHUNK
        pltpu.sync_copy(tself_ref.at[pl.ds(cN + r0, RCHUNK)], rowbuf)
        sgvec = sgv[pl.ds(k * RCHUNK, RCHUNK)]
        for i in range(RCHUNK):
            sg = sgvec[i]
            for t in range(8):
                sl = pl.ds(t * 16, 16)
                scbuf[i, sl] = scbuf[i, sl] * sg
        pltpu.sync_copy(rowbuf, agg.at[pl.ds(r0, RCHUNK)])
        return carry

    lax.fori_loop(0, nchunks, _ph1_body, 0)
    plsc.subcore_barrier()

    def _stage_group(g):
        row0 = s * NB + g * GB
        pltpu.sync_copy(sj_ref.at[pl.ds(row0, GB)], sj2)
        pltpu.sync_copy(ob_ref.at[pl.ds(row0, GB)], ob2)
        pltpu.sync_copy(pr_ref.at[pl.ds(row0, GB)], pr2)

    def _unpack_gate(packed, low_half):
        if low_half:
            bits = jnp.left_shift(packed, 16)
        else:
            bits = jnp.bitwise_and(packed, jnp.int32(-65536))
        return plsc.bitcast(bits, jnp.float32)

    def _direction(rb, src2, low_half, table_ref, dst2):
        def _gate_body(t, carry):
            sl = pl.ds(t * 16, 16)
            src16 = jnp.minimum(src2[rb, sl], N - 1)
            pr16 = pr2[rb, sl]
            adjv[sl] = src16 + cN
            packed = plsc.load_gather(g12v, [src16])
            x = _unpack_gate(packed, low_half) + plsc.load_gather(gbv, [pr16])
            gatev[sl] = _sigmoid16(x)
            return carry

        lax.fori_loop(0, NQ, _gate_body, 0)
        pltpu.async_copy(table_ref.at[adjv], msgp, sem).wait()

        def _scale_body(q, carry):
            e0 = q * 16
            gvec = gatev[pl.ds(e0, 16)]
            pvec = pr2[rb, pl.ds(e0, 16)]

            def _loads(i):
                p = pvec[i]
                mw = [msgp[e0 + i, pl.ds(u * 16, 16)] for u in range(4)]
                bw = [blabv[p, pl.ds(u * 16, 16)] for u in range(4)]
                return mw, bw

            cur = _loads(0)
            for i in range(16):
                nxt = _loads(i + 1) if i + 1 < 16 else None
                g = gvec[i]
                mw, bw = cur
                for t in range(8):
                    m = _unpack_gate(mw[t % 4], t < 4)
                    bl = _unpack_gate(bw[t % 4], t < 4)
                    scbuf[e0 + i, pl.ds(t * 16, 16)] = (m + bl) * g
                cur = nxt
            return carry

        lax.fori_loop(0, NQ, _scale_body, 0)
        pltpu.sync_copy(scbuf, agg.at[dst2.at[rb]], add=True)

    def _group_body(grp, carry):
        _stage_group(grp)

        def _pair_body(rb, carry2):
            _direction(rb, ob2, True, tsub_ref, sj2)
            _direction(rb, sj2, False, tobj_ref, ob2)
            return carry2

        lax.fori_loop(0, GB, _pair_body, 0)
        return carry

    lax.fori_loop(0, NB // GB, _group_body, 0)
    plsc.subcore_barrier()

    def _ph3_body(k, carry):
        r0 = base + k * RCHUNK
        pltpu.sync_copy(agg.at[pl.ds(r0, RCHUNK)], rowbuf)
        for i in range(RCHUNK):
            for t in range(8):
                sl = pl.ds(t * 16, 16)
                scbuf[i, sl] = jnp.maximum(scbuf[i, sl], 0.0)
        pltpu.sync_copy(rowbuf, out_ref.at[pl.ds(cN + r0, RCHUNK)])
        return carry

    lax.fori_loop(0, nchunks, _ph3_body, 0)


_SC_SCRATCH = [
    pltpu.VMEM_SHARED((N + 8, H), jnp.float32),
    pltpu.VMEM((GB, EB), jnp.int32),
    pltpu.VMEM((GB, EB), jnp.int32),
    pltpu.VMEM((GB, EB), jnp.int32),
    pltpu.VMEM((N,), jnp.int32),
    pltpu.VMEM((88, H // 2), jnp.int32),
    pltpu.VMEM((88,), jnp.float32),
    pltpu.VMEM((640,), jnp.float32),
    pltpu.VMEM((EB,), jnp.float32),
    pltpu.VMEM((EB,), jnp.int32),
    pltpu.VMEM((EB, H // 2), jnp.int32),
    pltpu.VMEM((EB, H), jnp.float32),
    pltpu.SemaphoreType.DMA,
]


def _sc_aggregate(tself_flat, tsub_flat, tobj_flat, g0, g12,
                  sj2d, ob2d, pr2d, blab2, gbp):
    mesh = plsc.VectorSubcoreMesh(core_axis_name="c", subcore_axis_name="s")
    fn = pl.kernel(
        _sc_body,
        out_type=jax.ShapeDtypeStruct((NC * N, H), jnp.float32),
        mesh=mesh,
        scratch_types=_SC_SCRATCH,
        compiler_params=pltpu.CompilerParams(needs_layout_passes=False,
                                             use_tc_tiling_on_sc=False),
    )
    return fn(tself_flat, tsub_flat, tobj_flat, g0, g12,
              sj2d, ob2d, pr2d, blab2, gbp)


def kernel(region_feats, rels, pred_classes, W_conv, W_g, b_lab, b_glab):
    wg_pad = jnp.zeros((D, H), jnp.float32).at[:, :3].set(W_g)
    tself_h, tsub_h, tobj_h, lgp = _tc_transform(region_feats, W_conv, wg_pad)

    tself_flat = tself_h.reshape(NC * N, H)
    tsub_flat = tsub_h.reshape(NC * N, H // 2)
    tobj_flat = tobj_h.reshape(NC * N, H // 2)
    g0 = lgp[:, 0]
    g1b = lax.bitcast_convert_type(lgp[:, 1].astype(jnp.bfloat16),
                                   jnp.uint16).astype(jnp.uint32)
    g2b = lax.bitcast_convert_type(lgp[:, 2].astype(jnp.bfloat16),
                                   jnp.uint16).astype(jnp.uint32)
    g12 = lax.bitcast_convert_type(g1b | (g2b << 16), jnp.int32)

    subj = rels[:, 1]
    obj = rels[:, 2]
    pad = EPAD - E
    padv = jnp.full((pad,), N, jnp.int32)
    sj2d = jnp.concatenate([subj, padv]).reshape(NS * NB, EB)
    ob2d = jnp.concatenate([obj, padv]).reshape(NS * NB, EB)
    pr2d = jnp.concatenate([pred_classes, jnp.zeros((pad,), jnp.int32)]
                           ).reshape(NS * NB, EB)
    def _pack_half(hm):
        lo = lax.bitcast_convert_type(hm[:, :64].astype(jnp.bfloat16),
                                      jnp.uint16).astype(jnp.uint32)
        hi = lax.bitcast_convert_type(hm[:, 64:].astype(jnp.bfloat16),
                                      jnp.uint16).astype(jnp.uint32)
        packed = lax.bitcast_convert_type(lo | (hi << 16), jnp.int32)
        return jnp.pad(packed, ((0, 7), (0, 0)))
    blab2 = jnp.concatenate([_pack_half(b_lab[:, :H]),
                             _pack_half(b_lab[:, H:])], axis=0)
    gbp = jnp.pad(b_glab[:, 0], (0, 7))

    out_flat = _sc_aggregate(tself_flat, tsub_flat, tobj_flat, g0, g12,
                             sj2d, ob2d, pr2d, blab2, gbp)
    out2 = out_flat.reshape(NC, N, H)
    return jnp.concatenate([out2[0], out2[1]], axis=1)


# speedup vs baseline: 1.9992x; 1.0353x over previous
import functools

import jax
import jax.numpy as jnp
from jax import lax
from jax.experimental import pallas as pl
from jax.experimental.pallas import tpu as pltpu
from jax.experimental.pallas import tpu_sc as plsc

N = 10000
D = 256
H = 128
E = 160000
P = 81
NC = 2
NS = 16
EB = 128
NB = 80
NQ = EB // 16
EPAD = NS * NB * EB
RT = 624
RCHUNK = 16


def _pack_cols(x):
    lo = lax.bitcast_convert_type(x[:, :64].astype(jnp.bfloat16),
                                  jnp.uint16).astype(jnp.uint32)
    hi = lax.bitcast_convert_type(x[:, 64:].astype(jnp.bfloat16),
                                  jnp.uint16).astype(jnp.uint32)
    return lax.bitcast_convert_type(lo | (hi << 16), jnp.int32)


def _tc_body(rf_ref, wc_ref, wg_ref, tself_ref, tsub_ref, tobj_ref, lgp_ref):
    rf = rf_ref[...]
    t = jnp.dot(rf, wc_ref[...], preferred_element_type=jnp.float32)
    lgp_ref[...] = jnp.dot(rf, wg_ref[...], preferred_element_type=jnp.float32)
    tself_ref[0] = t[:, 0:H]
    tself_ref[1] = t[:, H:2 * H]
    tsub_ref[0] = _pack_cols(t[:, 2 * H:3 * H])
    tsub_ref[1] = _pack_cols(t[:, 3 * H:4 * H])
    tobj_ref[0] = _pack_cols(t[:, 4 * H:5 * H])
    tobj_ref[1] = _pack_cols(t[:, 5 * H:6 * H])


def _tc_transform(region_feats, W_conv, wg_pad):
    R = 400
    return pl.pallas_call(
        _tc_body,
        grid=(N // R,),
        in_specs=[
            pl.BlockSpec((R, D), lambda i: (i, 0)),
            pl.BlockSpec((D, 3 * D), lambda i: (0, 0)),
            pl.BlockSpec((D, H), lambda i: (0, 0)),
        ],
        out_specs=[
            pl.BlockSpec((NC, R, H), lambda i: (0, i, 0)),
            pl.BlockSpec((NC, R, H // 2), lambda i: (0, i, 0)),
            pl.BlockSpec((NC, R, H // 2), lambda i: (0, i, 0)),
            pl.BlockSpec((R, H), lambda i: (i, 0)),
        ],
        out_shape=[
            jax.ShapeDtypeStruct((NC, N, H), jnp.float32),
            jax.ShapeDtypeStruct((NC, N, H // 2), jnp.int32),
            jax.ShapeDtypeStruct((NC, N, H // 2), jnp.int32),
            jax.ShapeDtypeStruct((N, H), jnp.float32),
        ],
    )(region_feats, W_conv, wg_pad)


def _sigmoid16(x):
    return 1.0 / (1.0 + jnp.exp(-x))


GB = 8


def _sc_body(tself_ref, tsub_ref, tobj_ref, g0_ref, g12_ref,
             sj_ref, ob_ref, pr_ref, blab_ref, gb_ref,
             out_ref,
             agg, sj2, ob2, pr2, g12v, blabv, gbv, sgv,
             gatev, adjv, msgp, scbuf, sem):
    c = lax.axis_index("c")
    s = lax.axis_index("s")
    cN = c * N

    pltpu.sync_copy(g12_ref, g12v)
    pltpu.sync_copy(gb_ref, gbv)
    pltpu.sync_copy(blab_ref.at[pl.ds(c * 88, 88)], blabv)

    base = s * RT
    nchunks = jnp.where(s == NS - 1, 40, 39)

    pltpu.sync_copy(g0_ref.at[pl.ds(base, 640)], sgv)

    def _sig_body(k, carry):
        x = sgv[pl.ds(k * 16, 16)]
        sgv[pl.ds(k * 16, 16)] = _sigmoid16(x)
        return carry

    lax.fori_loop(0, 40, _sig_body, 0)

    rowbuf = scbuf.at[pl.ds(0, RCHUNK)]

    def _ph1_body(k, carry):
        r0 = base + k * RCHUNK
        pltpu.sync_copy(tself_ref.at[pl.ds(cN + r0, RCHUNK)], rowbuf)
        sgvec = sgv[pl.ds(k * RCHUNK, RCHUNK)]
        for i in range(RCHUNK):
            sg = sgvec[i]
            for t in range(8):
                sl = pl.ds(t * 16, 16)
                scbuf[i, sl] = scbuf[i, sl] * sg
        pltpu.sync_copy(rowbuf, agg.at[pl.ds(r0, RCHUNK)])
        return carry

    lax.fori_loop(0, nchunks, _ph1_body, 0)
    plsc.subcore_barrier()

    def _stage_group(g):
        row0 = s * NB + g * GB
        pltpu.sync_copy(sj_ref.at[pl.ds(row0, GB)], sj2)
        pltpu.sync_copy(ob_ref.at[pl.ds(row0, GB)], ob2)
        pltpu.sync_copy(pr_ref.at[pl.ds(row0, GB)], pr2)

    def _unpack_gate(packed, low_half):
        if low_half:
            bits = jnp.left_shift(packed, 16)
        else:
            bits = jnp.bitwise_and(packed, jnp.int32(-65536))
        return plsc.bitcast(bits, jnp.float32)

    def _direction(rb, src2, low_half, table_ref, dst2):
        def _adj_body(t, carry):
            sl = pl.ds(t * 16, 16)
            adjv[sl] = jnp.minimum(src2[rb, sl], N - 1) + cN
            return carry

        lax.fori_loop(0, NQ, _adj_body, 0)
        pltpu.async_copy(table_ref.at[adjv], msgp, sem)

        def _gate_body(t, carry):
            sl = pl.ds(t * 16, 16)
            src16 = jnp.minimum(src2[rb, sl], N - 1)
            pr16 = pr2[rb, sl]
            packed = plsc.load_gather(g12v, [src16])
            x = _unpack_gate(packed, low_half) + plsc.load_gather(gbv, [pr16])
            gatev[sl] = _sigmoid16(x)
            return carry

        lax.fori_loop(0, NQ, _gate_body, 0)
        pltpu.make_async_copy(table_ref.at[adjv], msgp, sem).wait()

        def _scale_body(q, carry):
            e0 = q * 16
            gvec = gatev[pl.ds(e0, 16)]
            pvec = pr2[rb, pl.ds(e0, 16)]

            def _loads(i):
                p = pvec[i]
                mw = [msgp[e0 + i, pl.ds(u * 16, 16)] for u in range(4)]
                bw = [blabv[p, pl.ds(u * 16, 16)] for u in range(4)]
                return mw, bw

            cur = _loads(0)
            for i in range(16):
                nxt = _loads(i + 1) if i + 1 < 16 else None
                g = gvec[i]
                mw, bw = cur
                for t in range(8):
                    m = _unpack_gate(mw[t % 4], t < 4)
                    bl = _unpack_gate(bw[t % 4], t < 4)
                    scbuf[e0 + i, pl.ds(t * 16, 16)] = (m + bl) * g
                cur = nxt
            return carry

        lax.fori_loop(0, NQ, _scale_body, 0)
        pltpu.sync_copy(scbuf, agg.at[dst2.at[rb]], add=True)

    def _group_body(grp, carry):
        _stage_group(grp)

        def _pair_body(rb, carry2):
            _direction(rb, ob2, True, tsub_ref, sj2)
            _direction(rb, sj2, False, tobj_ref, ob2)
            return carry2

        lax.fori_loop(0, GB, _pair_body, 0)
        return carry

    lax.fori_loop(0, NB // GB, _group_body, 0)
    plsc.subcore_barrier()

    def _ph3_body(k, carry):
        r0 = base + k * RCHUNK
        pltpu.sync_copy(agg.at[pl.ds(r0, RCHUNK)], rowbuf)
        for i in range(RCHUNK):
            for t in range(8):
                sl = pl.ds(t * 16, 16)
                scbuf[i, sl] = jnp.maximum(scbuf[i, sl], 0.0)
        pltpu.sync_copy(rowbuf, out_ref.at[pl.ds(cN + r0, RCHUNK)])
        return carry

    lax.fori_loop(0, nchunks, _ph3_body, 0)


_SC_SCRATCH = [
    pltpu.VMEM_SHARED((N + 8, H), jnp.float32),
    pltpu.VMEM((GB, EB), jnp.int32),
    pltpu.VMEM((GB, EB), jnp.int32),
    pltpu.VMEM((GB, EB), jnp.int32),
    pltpu.VMEM((N,), jnp.int32),
    pltpu.VMEM((88, H // 2), jnp.int32),
    pltpu.VMEM((88,), jnp.float32),
    pltpu.VMEM((640,), jnp.float32),
    pltpu.VMEM((EB,), jnp.float32),
    pltpu.VMEM((EB,), jnp.int32),
    pltpu.VMEM((EB, H // 2), jnp.int32),
    pltpu.VMEM((EB, H), jnp.float32),
    pltpu.SemaphoreType.DMA,
]


def _sc_aggregate(tself_flat, tsub_flat, tobj_flat, g0, g12,
                  sj2d, ob2d, pr2d, blab2, gbp):
    mesh = plsc.VectorSubcoreMesh(core_axis_name="c", subcore_axis_name="s")
    fn = pl.kernel(
        _sc_body,
        out_type=jax.ShapeDtypeStruct((NC * N, H), jnp.float32),
        mesh=mesh,
        scratch_types=_SC_SCRATCH,
        compiler_params=pltpu.CompilerParams(needs_layout_passes=False,
                                             use_tc_tiling_on_sc=False),
    )
    return fn(tself_flat, tsub_flat, tobj_flat, g0, g12,
              sj2d, ob2d, pr2d, blab2, gbp)


def kernel(region_feats, rels, pred_classes, W_conv, W_g, b_lab, b_glab):
    wg_pad = jnp.zeros((D, H), jnp.float32).at[:, :3].set(W_g)
    tself_h, tsub_h, tobj_h, lgp = _tc_transform(region_feats, W_conv, wg_pad)

    tself_flat = tself_h.reshape(NC * N, H)
    tsub_flat = tsub_h.reshape(NC * N, H // 2)
    tobj_flat = tobj_h.reshape(NC * N, H // 2)
    g0 = lgp[:, 0]
    g1b = lax.bitcast_convert_type(lgp[:, 1].astype(jnp.bfloat16),
                                   jnp.uint16).astype(jnp.uint32)
    g2b = lax.bitcast_convert_type(lgp[:, 2].astype(jnp.bfloat16),
                                   jnp.uint16).astype(jnp.uint32)
    g12 = lax.bitcast_convert_type(g1b | (g2b << 16), jnp.int32)

    subj = rels[:, 1]
    obj = rels[:, 2]
    pad = EPAD - E
    padv = jnp.full((pad,), N, jnp.int32)
    sj2d = jnp.concatenate([subj, padv]).reshape(NS * NB, EB)
    ob2d = jnp.concatenate([obj, padv]).reshape(NS * NB, EB)
    pr2d = jnp.concatenate([pred_classes, jnp.zeros((pad,), jnp.int32)]
                           ).reshape(NS * NB, EB)
    def _pack_half(hm):
        lo = lax.bitcast_convert_type(hm[:, :64].astype(jnp.bfloat16),
                                      jnp.uint16).astype(jnp.uint32)
        hi = lax.bitcast_convert_type(hm[:, 64:].astype(jnp.bfloat16),
                                      jnp.uint16).astype(jnp.uint32)
        packed = lax.bitcast_convert_type(lo | (hi << 16), jnp.int32)
        return jnp.pad(packed, ((0, 7), (0, 0)))
    blab2 = jnp.concatenate([_pack_half(b_lab[:, :H]),
                             _pack_half(b_lab[:, H:])], axis=0)
    gbp = jnp.pad(b_glab[:, 0], (0, 7))

    out_flat = _sc_aggregate(tself_flat, tsub_flat, tobj_flat, g0, g12,
                             sj2d, ob2d, pr2d, blab2, gbp)
    out2 = out_flat.reshape(NC, N, H)
    return jnp.concatenate([out2[0], out2[1]], axis=1)
